# baseline (speedup 1.0000x reference)
import functools

import jax
import jax.numpy as jnp
from jax import lax
from jax.experimental import pallas as pl
from jax.experimental.pallas import tpu as pltpu
from jax.experimental.pallas import tpu_sc as plsc

_LANES = 16
_MIN_I32 = -2147483648
_RIC = 2


def _make_sc_numerators(n_cols, n_rows, num_workers, interpret=False):
    seglen = n_cols // _LANES
    rows_per = n_rows // num_workers
    tmask = seglen - 1
    tshift = seglen.bit_length() - 1
    radix = 256
    mu = (n_cols - 1) / 2.0

    mesh = plsc.VectorSubcoreMesh(core_axis_name="c", subcore_axis_name="s",
                                  num_cores=2, num_subcores=16)

    row_scratch = [
        pltpu.VMEM((n_cols,), jnp.float32),
        pltpu.VMEM((n_cols,), jnp.float32),
        pltpu.VMEM((n_cols,), jnp.int32),
        pltpu.VMEM((n_cols,), jnp.int32),
        pltpu.VMEM((n_cols,), jnp.int32),
        pltpu.VMEM((n_cols,), jnp.int32),
        pltpu.VMEM((radix * _LANES,), jnp.int32),
    ]

    @functools.partial(
        pl.kernel,
        out_type=jax.ShapeDtypeStruct((n_rows * _LANES,), jnp.float32),
        mesh=mesh,
        scratch_types=row_scratch * _RIC + [
            pltpu.VMEM((rows_per * _LANES,), jnp.float32),
        ],
        compiler_params=pltpu.CompilerParams(needs_layout_passes=False),
        interpret=interpret,
    )
    def sc_kernel(pred_hbm, targ_hbm, out_hbm, *scr):
        bufps = [scr[j * 7 + 0] for j in range(_RIC)]
        bufts = [scr[j * 7 + 1] for j in range(_RIC)]
        kas = [scr[j * 7 + 2] for j in range(_RIC)]
        kbs = [scr[j * 7 + 3] for j in range(_RIC)]
        pas = [scr[j * 7 + 4] for j in range(_RIC)]
        pbs = [scr[j * 7 + 5] for j in range(_RIC)]
        hists = [scr[j * 7 + 6] for j in range(_RIC)]
        nums = scr[_RIC * 7]

        wid = lax.axis_index("s") * 2 + lax.axis_index("c")
        lane = lax.iota(jnp.int32, _LANES)
        zeros16 = jnp.zeros((_LANES,), jnp.int32)
        ones16 = jnp.ones((_LANES,), jnp.int32)

        def skew(i):
            t = i & tmask
            s = jnp.right_shift(i, tshift)
            return (t << 4) + ((s + t) & 15)

        def key_transpose(srcs, dsts):
            def body(g, _):
                i = g * _LANES + lane
                for j in range(_RIC):
                    x = srcs[j][pl.ds(g * _LANES, _LANES)]
                    b = plsc.bitcast(x, jnp.int32)
                    b = jnp.where(x == 0.0, 0, b)
                    m = jnp.right_shift(b, 31)
                    key = (b ^ (m & 0x7FFFFFFF)) ^ _MIN_I32
                    plsc.store_scatter(dsts[j], [skew(i)], key)
                return 0
            lax.fori_loop(0, seglen, body, 0, unroll=4)

        def radix_pass(shift, ksrcs, psrcs, kdsts, pdsts,
                       gen_payload=False, invert_to=None, dot=False):
            def clr(d, _):
                for j in range(_RIC):
                    hists[j][pl.ds(d * _LANES, _LANES)] = zeros16
                return 0
            lax.fori_loop(0, radix, clr, 0, unroll=8)

            def histo(t, _):
                s = (lane - t) & 15
                for j in range(_RIC):
                    k = ksrcs[j][pl.ds(t * _LANES, _LANES)]
                    digit = jnp.right_shift(k, shift) & 255
                    plsc.addupdate_scatter(hists[j], [(digit << 4) + s], ones16)
                return 0
            lax.fori_loop(0, seglen, histo, 0, unroll=4)

            def scan(d, tots):
                new = []
                for j in range(_RIC):
                    h = hists[j][pl.ds(d * _LANES, _LANES)]
                    incl = plsc.cumsum(h)
                    hists[j][pl.ds(d * _LANES, _LANES)] = incl - h + tots[j]
                    new.append(tots[j] + jnp.sum(h))
                return tuple(new)
            lax.fori_loop(0, radix, scan, (jnp.int32(0),) * _RIC, unroll=4)

            if dot:
                def permute_dot(t, accs):
                    s = (lane - t) & 15
                    new = []
                    for j in range(_RIC):
                        k = ksrcs[j][pl.ds(t * _LANES, _LANES)]
                        digit = jnp.right_shift(k, shift) & 255
                        hidx = (digit << 4) + s
                        pos = plsc.load_gather(hists[j], [hidx])
                        plsc.store_scatter(hists[j], [hidx], pos + 1)
                        payload = psrcs[j][pl.ds(t * _LANES, _LANES)]
                        fm = pos.astype(jnp.float32) - mu
                        fv = payload.astype(jnp.float32) - mu
                        new.append(accs[j] + fm * fv)
                    return tuple(new)
                return lax.fori_loop(
                    0, seglen, permute_dot,
                    (jnp.zeros((_LANES,), jnp.float32),) * _RIC, unroll=2)

            def permute(t, _):
                s = (lane - t) & 15
                for j in range(_RIC):
                    k = ksrcs[j][pl.ds(t * _LANES, _LANES)]
                    digit = jnp.right_shift(k, shift) & 255
                    hidx = (digit << 4) + s
                    pos = plsc.load_gather(hists[j], [hidx])
                    plsc.store_scatter(hists[j], [hidx], pos + 1)
                    if gen_payload:
                        payload = (s << tshift) + t
                    else:
                        payload = psrcs[j][pl.ds(t * _LANES, _LANES)]
                    if invert_to is not None:
                        plsc.store_scatter(invert_to[j], [skew(payload)], pos)
                    else:
                        dest = skew(pos)
                        plsc.store_scatter(kdsts[j], [dest], k)
                        plsc.store_scatter(pdsts[j], [dest], payload)
                return 0
            lax.fori_loop(0, seglen, permute, 0, unroll=2)
            return None

        def do_pair(r, _):
            for j in range(_RIC):
                row = wid * rows_per + r * _RIC + j
                pltpu.sync_copy(pred_hbm.at[row], bufps[j])
                pltpu.sync_copy(targ_hbm.at[row], bufts[j])
            key_transpose(bufps, kas)
            radix_pass(0, kas, pas, kbs, pbs, gen_payload=True)
            radix_pass(8, kbs, pbs, kas, pas)
            radix_pass(16, kas, pas, kbs, pbs)
            radix_pass(24, kbs, pbs, None, None, invert_to=pas)
            key_transpose(bufts, kas)
            radix_pass(0, kas, pas, kbs, pbs)
            radix_pass(8, kbs, pbs, kas, pas)
            radix_pass(16, kas, pas, kbs, pbs)
            accs = radix_pass(24, kbs, pbs, None, None, dot=True)
            for j in range(_RIC):
                nums[pl.ds((r * _RIC + j) * _LANES, _LANES)] = accs[j]
            return 0

        lax.fori_loop(0, rows_per // _RIC, do_pair, 0)
        pltpu.sync_copy(
            nums, out_hbm.at[pl.ds(wid * rows_per * _LANES, rows_per * _LANES)])

    return sc_kernel


def _tc_reduce(nums, scale):
    n = nums.shape[0]
    x2d = nums.reshape(n // 128, 128)

    def body(x_ref, o_ref):
        o_ref[0, 0] = jnp.sum(x_ref[...]) * jnp.float32(scale)

    out = pl.pallas_call(
        body,
        out_shape=jax.ShapeDtypeStruct((1, 1), jnp.float32),
        in_specs=[pl.BlockSpec(memory_space=pltpu.VMEM)],
        out_specs=pl.BlockSpec(memory_space=pltpu.SMEM),
    )(x2d)
    return out[0, 0]


def kernel(predictions, targets):
    n_rows, n_cols = predictions.shape
    var = float(n_cols) * (float(n_cols) ** 2 - 1.0) / 12.0
    scale = -1.0 / ((var + 1e-8) * n_rows)
    sc = _make_sc_numerators(n_cols, n_rows, 32)
    nums = sc(predictions, targets)
    return _tc_reduce(nums, scale)


# speedup vs baseline: 1.8238x; 1.8238x over previous
import functools

import jax
import jax.numpy as jnp
from jax import lax
from jax.experimental import pallas as pl
from jax.experimental.pallas import tpu as pltpu
from jax.experimental.pallas import tpu_sc as plsc

_LANES = 16
_MIN_I32 = -2147483648
_RIC = 2


def _make_sc_numerators(n_cols, n_rows, num_workers, interpret=False):
    seglen = n_cols // _LANES
    rows_per = n_rows // num_workers
    tmask = seglen - 1
    tshift = seglen.bit_length() - 1
    radix = 256
    mu = (n_cols - 1) / 2.0

    mesh = plsc.VectorSubcoreMesh(core_axis_name="c", subcore_axis_name="s",
                                  num_cores=2, num_subcores=16)

    row_scratch = [
        pltpu.VMEM((n_cols,), jnp.float32),
        pltpu.VMEM((n_cols,), jnp.float32),
        pltpu.VMEM((n_cols,), jnp.int32),
        pltpu.VMEM((n_cols,), jnp.int32),
        pltpu.VMEM((n_cols,), jnp.int32),
        pltpu.VMEM((n_cols,), jnp.int32),
        pltpu.VMEM((radix * _LANES,), jnp.int32),
        pltpu.VMEM((_LANES,), jnp.int32),
    ]

    @functools.partial(
        pl.kernel,
        out_type=jax.ShapeDtypeStruct((n_rows * _LANES,), jnp.float32),
        mesh=mesh,
        scratch_types=row_scratch * _RIC + [
            pltpu.VMEM((rows_per * _LANES,), jnp.float32),
        ],
        compiler_params=pltpu.CompilerParams(needs_layout_passes=False),
        interpret=interpret,
    )
    def sc_kernel(pred_hbm, targ_hbm, out_hbm, *scr):
        bufps = [scr[j * 8 + 0] for j in range(_RIC)]
        bufts = [scr[j * 8 + 1] for j in range(_RIC)]
        kas = [scr[j * 8 + 2] for j in range(_RIC)]
        kbs = [scr[j * 8 + 3] for j in range(_RIC)]
        pas = [scr[j * 8 + 4] for j in range(_RIC)]
        pbs = [scr[j * 8 + 5] for j in range(_RIC)]
        hists = [scr[j * 8 + 6] for j in range(_RIC)]
        scrs = [scr[j * 8 + 7] for j in range(_RIC)]
        nums = scr[_RIC * 8]

        wid = lax.axis_index("s") * 2 + lax.axis_index("c")
        lane = lax.iota(jnp.int32, _LANES)
        zeros16 = jnp.zeros((_LANES,), jnp.int32)
        ones16 = jnp.ones((_LANES,), jnp.int32)

        def skew(i):
            t = i & tmask
            s = jnp.right_shift(i, tshift)
            return (t << 4) + ((s + t) & 15)

        def key_transpose(srcs, dsts):
            def body(g, _):
                i = g * _LANES + lane
                dest = skew(i)
                xs = [srcs[j][pl.ds(g * _LANES, _LANES)] for j in range(_RIC)]
                keys = []
                for x in xs:
                    b = plsc.bitcast(x, jnp.int32)
                    b = jnp.where(x == 0.0, 0, b)
                    m = jnp.right_shift(b, 31)
                    keys.append((b ^ (m & 0x7FFFFFFF)) ^ _MIN_I32)
                for j in range(_RIC):
                    plsc.store_scatter(dsts[j], [dest], keys[j])
                return 0
            lax.fori_loop(0, seglen, body, 0, unroll=4)

        def radix_pass(shift, ksrcs, psrcs, kdsts, pdsts,
                       gen_payload=False, invert_to=None, dot=False):
            def clr(d, _):
                for j in range(_RIC):
                    hists[j][pl.ds(d * _LANES, _LANES)] = zeros16
                return 0
            lax.fori_loop(0, radix, clr, 0, unroll=8)

            def histo(t, _):
                s = (lane - t) & 15
                ks = [ksrcs[j][pl.ds(t * _LANES, _LANES)] for j in range(_RIC)]
                hidxs = [((jnp.right_shift(k, shift) & 255) << 4) + s for k in ks]
                for j in range(_RIC):
                    plsc.addupdate_scatter(hists[j], [hidxs[j]], ones16)
                return 0
            lax.fori_loop(0, seglen, histo, 0, unroll=8)

            def scan(d, tots):
                hs = [hists[j][pl.ds(d * _LANES, _LANES)] for j in range(_RIC)]
                incls = [plsc.cumsum(h) for h in hs]
                for j in range(_RIC):
                    hists[j][pl.ds(d * _LANES, _LANES)] = incls[j] - hs[j] + tots[j]
                return tuple(tots[j] + incls[j][_LANES - 1] for j in range(_RIC))
            lax.fori_loop(0, radix, scan, (jnp.int32(0),) * _RIC, unroll=4)

            if dot:
                def permute_dot(t, accs):
                    s = (lane - t) & 15
                    ks = [ksrcs[j][pl.ds(t * _LANES, _LANES)] for j in range(_RIC)]
                    pays = [psrcs[j][pl.ds(t * _LANES, _LANES)] for j in range(_RIC)]
                    hidxs = [((jnp.right_shift(k, shift) & 255) << 4) + s
                             for k in ks]
                    poss = [plsc.load_gather(hists[j], [hidxs[j]])
                            for j in range(_RIC)]
                    for j in range(_RIC):
                        plsc.store_scatter(hists[j], [hidxs[j]], poss[j] + 1)
                    new = []
                    for j in range(_RIC):
                        fm = poss[j].astype(jnp.float32) - mu
                        fv = pays[j].astype(jnp.float32) - mu
                        new.append(accs[j] + fm * fv)
                    return tuple(new)
                return lax.fori_loop(
                    0, seglen, permute_dot,
                    (jnp.zeros((_LANES,), jnp.float32),) * _RIC, unroll=4)

            def permute(t, _):
                s = (lane - t) & 15
                ks = [ksrcs[j][pl.ds(t * _LANES, _LANES)] for j in range(_RIC)]
                if gen_payload:
                    pays = [(s << tshift) + t] * _RIC
                else:
                    pays = [psrcs[j][pl.ds(t * _LANES, _LANES)]
                            for j in range(_RIC)]
                hidxs = [((jnp.right_shift(k, shift) & 255) << 4) + s
                         for k in ks]
                poss = [plsc.load_gather(hists[j], [hidxs[j]])
                        for j in range(_RIC)]
                for j in range(_RIC):
                    plsc.store_scatter(hists[j], [hidxs[j]], poss[j] + 1)
                if invert_to is not None:
                    dests = [skew(pays[j]) for j in range(_RIC)]
                    for j in range(_RIC):
                        plsc.store_scatter(invert_to[j], [dests[j]], poss[j])
                else:
                    dests = [skew(poss[j]) for j in range(_RIC)]
                    for j in range(_RIC):
                        plsc.store_scatter(kdsts[j], [dests[j]], ks[j])
                    for j in range(_RIC):
                        plsc.store_scatter(pdsts[j], [dests[j]], pays[j])
                return 0
            lax.fori_loop(0, seglen, permute, 0, unroll=4)
            return None

        def do_pair(r, _):
            for j in range(_RIC):
                row = wid * rows_per + r * _RIC + j
                pltpu.sync_copy(pred_hbm.at[row], bufps[j])
                pltpu.sync_copy(targ_hbm.at[row], bufts[j])
            key_transpose(bufps, kas)
            radix_pass(0, kas, pas, kbs, pbs, gen_payload=True)
            radix_pass(8, kbs, pbs, kas, pas)
            radix_pass(16, kas, pas, kbs, pbs)
            radix_pass(24, kbs, pbs, None, None, invert_to=pas)
            key_transpose(bufts, kas)
            radix_pass(0, kas, pas, kbs, pbs)
            radix_pass(8, kbs, pbs, kas, pas)
            radix_pass(16, kas, pas, kbs, pbs)
            accs = radix_pass(24, kbs, pbs, None, None, dot=True)
            for j in range(_RIC):
                nums[pl.ds((r * _RIC + j) * _LANES, _LANES)] = accs[j]
            return 0

        lax.fori_loop(0, rows_per // _RIC, do_pair, 0)
        pltpu.sync_copy(
            nums, out_hbm.at[pl.ds(wid * rows_per * _LANES, rows_per * _LANES)])

    return sc_kernel


def _tc_reduce(nums, scale):
    n = nums.shape[0]
    x2d = nums.reshape(n // 128, 128)

    def body(x_ref, o_ref):
        o_ref[0, 0] = jnp.sum(x_ref[...]) * jnp.float32(scale)

    out = pl.pallas_call(
        body,
        out_shape=jax.ShapeDtypeStruct((1, 1), jnp.float32),
        in_specs=[pl.BlockSpec(memory_space=pltpu.VMEM)],
        out_specs=pl.BlockSpec(memory_space=pltpu.SMEM),
    )(x2d)
    return out[0, 0]


def kernel(predictions, targets):
    n_rows, n_cols = predictions.shape
    var = float(n_cols) * (float(n_cols) ** 2 - 1.0) / 12.0
    scale = -1.0 / ((var + 1e-8) * n_rows)
    sc = _make_sc_numerators(n_cols, n_rows, 32)
    nums = sc(predictions, targets)
    return _tc_reduce(nums, scale)


# speedup vs baseline: 3.0498x; 1.6722x over previous
import functools

import jax
import jax.numpy as jnp
from jax import lax
from jax.experimental import pallas as pl
from jax.experimental.pallas import tpu as pltpu
from jax.experimental.pallas import tpu_sc as plsc

_LANES = 16
_MIN_I32 = -2147483648
_RIC = 4


def _make_sc_numerators(n_cols, n_rows, num_workers, interpret=False):
    seglen = n_cols // _LANES
    rows_per = n_rows // num_workers
    tmask = seglen - 1
    tshift = seglen.bit_length() - 1
    radix = 256
    mu = (n_cols - 1) / 2.0

    mesh = plsc.VectorSubcoreMesh(core_axis_name="c", subcore_axis_name="s",
                                  num_cores=2, num_subcores=16)

    row_scratch = [
        pltpu.VMEM((n_cols,), jnp.float32),
        pltpu.VMEM((n_cols,), jnp.float32),
        pltpu.VMEM((n_cols,), jnp.int32),
        pltpu.VMEM((n_cols,), jnp.int32),
        pltpu.VMEM((n_cols,), jnp.int32),
        pltpu.VMEM((n_cols,), jnp.int32),
        pltpu.VMEM((radix * _LANES,), jnp.int32),
    ]

    @functools.partial(
        pl.kernel,
        out_type=jax.ShapeDtypeStruct((n_rows * _LANES,), jnp.float32),
        mesh=mesh,
        scratch_types=row_scratch * _RIC + [
            pltpu.VMEM((rows_per * _LANES,), jnp.float32),
            pltpu.SemaphoreType.DMA,
        ],
        compiler_params=pltpu.CompilerParams(needs_layout_passes=False),
        interpret=interpret,
    )
    def sc_kernel(pred_hbm, targ_hbm, out_hbm, *scr):
        bufps = [scr[j * 7 + 0] for j in range(_RIC)]
        bufts = [scr[j * 7 + 1] for j in range(_RIC)]
        kas = [scr[j * 7 + 2] for j in range(_RIC)]
        kbs = [scr[j * 7 + 3] for j in range(_RIC)]
        pas = [scr[j * 7 + 4] for j in range(_RIC)]
        pbs = [scr[j * 7 + 5] for j in range(_RIC)]
        hists = [scr[j * 7 + 6] for j in range(_RIC)]
        nums = scr[_RIC * 7]
        dma_sem = scr[_RIC * 7 + 1]

        wid = lax.axis_index("s") * 2 + lax.axis_index("c")
        lane = lax.iota(jnp.int32, _LANES)
        zeros16 = jnp.zeros((_LANES,), jnp.int32)
        ones16 = jnp.ones((_LANES,), jnp.int32)

        def skew(i):
            t = i & tmask
            s = jnp.right_shift(i, tshift)
            return (t << 4) + ((s + t) & 15)

        def key_transpose(srcs, dsts):
            def body(g, _):
                i = g * _LANES + lane
                dest = skew(i)
                xs = [srcs[j][pl.ds(g * _LANES, _LANES)] for j in range(_RIC)]
                keys = []
                for x in xs:
                    b = plsc.bitcast(x, jnp.int32)
                    b = jnp.where(x == 0.0, 0, b)
                    m = jnp.right_shift(b, 31)
                    keys.append((b ^ (m & 0x7FFFFFFF)) ^ _MIN_I32)
                for j in range(_RIC):
                    plsc.store_scatter(dsts[j], [dest], keys[j])
                return 0
            lax.fori_loop(0, seglen, body, 0, unroll=2)

        def radix_pass(shift, ksrcs, psrcs, kdsts, pdsts,
                       gen_payload=False, invert_to=None, dot=False):
            def clr(d, _):
                for j in range(_RIC):
                    hists[j][pl.ds(d * _LANES, _LANES)] = zeros16
                return 0
            lax.fori_loop(0, radix, clr, 0, unroll=8)

            def histo(t, _):
                s = (lane - t) & 15
                ks = [ksrcs[j][pl.ds(t * _LANES, _LANES)] for j in range(_RIC)]
                hidxs = [((jnp.right_shift(k, shift) & 255) << 4) + s for k in ks]
                for j in range(_RIC):
                    plsc.addupdate_scatter(hists[j], [hidxs[j]], ones16)
                return 0
            lax.fori_loop(0, seglen, histo, 0, unroll=4)

            def scan(d, tots):
                hs = [hists[j][pl.ds(d * _LANES, _LANES)] for j in range(_RIC)]
                incls = [plsc.cumsum(h) for h in hs]
                for j in range(_RIC):
                    hists[j][pl.ds(d * _LANES, _LANES)] = incls[j] - hs[j] + tots[j]
                return tuple(tots[j] + incls[j][_LANES - 1] for j in range(_RIC))
            lax.fori_loop(0, radix, scan, (jnp.int32(0),) * _RIC, unroll=2)

            if dot:
                def permute_dot(t, accs):
                    s = (lane - t) & 15
                    ks = [ksrcs[j][pl.ds(t * _LANES, _LANES)] for j in range(_RIC)]
                    pays = [psrcs[j][pl.ds(t * _LANES, _LANES)] for j in range(_RIC)]
                    hidxs = [((jnp.right_shift(k, shift) & 255) << 4) + s
                             for k in ks]
                    poss = [plsc.load_gather(hists[j], [hidxs[j]])
                            for j in range(_RIC)]
                    for j in range(_RIC):
                        plsc.store_scatter(hists[j], [hidxs[j]], poss[j] + 1)
                    new = []
                    for j in range(_RIC):
                        fm = poss[j].astype(jnp.float32) - mu
                        fv = pays[j].astype(jnp.float32) - mu
                        new.append(accs[j] + fm * fv)
                    return tuple(new)
                return lax.fori_loop(
                    0, seglen, permute_dot,
                    (jnp.zeros((_LANES,), jnp.float32),) * _RIC, unroll=2)

            def permute(t, _):
                s = (lane - t) & 15
                ks = [ksrcs[j][pl.ds(t * _LANES, _LANES)] for j in range(_RIC)]
                if gen_payload:
                    pays = [(s << tshift) + t] * _RIC
                else:
                    pays = [psrcs[j][pl.ds(t * _LANES, _LANES)]
                            for j in range(_RIC)]
                hidxs = [((jnp.right_shift(k, shift) & 255) << 4) + s
                         for k in ks]
                poss = [plsc.load_gather(hists[j], [hidxs[j]])
                        for j in range(_RIC)]
                for j in range(_RIC):
                    plsc.store_scatter(hists[j], [hidxs[j]], poss[j] + 1)
                if invert_to is not None:
                    dests = [skew(pays[j]) for j in range(_RIC)]
                    for j in range(_RIC):
                        plsc.store_scatter(invert_to[j], [dests[j]], poss[j])
                else:
                    dests = [skew(poss[j]) for j in range(_RIC)]
                    for j in range(_RIC):
                        plsc.store_scatter(kdsts[j], [dests[j]], ks[j])
                    for j in range(_RIC):
                        plsc.store_scatter(pdsts[j], [dests[j]], pays[j])
                return 0
            lax.fori_loop(0, seglen, permute, 0, unroll=2)
            return None

        def do_pair(r, _):
            copies = []
            for j in range(_RIC):
                row = wid * rows_per + r * _RIC + j
                copies.append(pltpu.async_copy(pred_hbm.at[row], bufps[j], dma_sem))
                copies.append(pltpu.async_copy(targ_hbm.at[row], bufts[j], dma_sem))
            for c in copies:
                c.wait()
            key_transpose(bufps, kas)
            radix_pass(0, kas, pas, kbs, pbs, gen_payload=True)
            radix_pass(8, kbs, pbs, kas, pas)
            radix_pass(16, kas, pas, kbs, pbs)
            radix_pass(24, kbs, pbs, None, None, invert_to=pas)
            key_transpose(bufts, kas)
            radix_pass(0, kas, pas, kbs, pbs)
            radix_pass(8, kbs, pbs, kas, pas)
            radix_pass(16, kas, pas, kbs, pbs)
            accs = radix_pass(24, kbs, pbs, None, None, dot=True)
            for j in range(_RIC):
                nums[pl.ds((r * _RIC + j) * _LANES, _LANES)] = accs[j]
            return 0

        lax.fori_loop(0, rows_per // _RIC, do_pair, 0)
        pltpu.sync_copy(
            nums, out_hbm.at[pl.ds(wid * rows_per * _LANES, rows_per * _LANES)])

    return sc_kernel


def _tc_reduce(nums, scale):
    n = nums.shape[0]
    x2d = nums.reshape(n // 128, 128)

    def body(x_ref, o_ref):
        o_ref[0, 0] = jnp.sum(x_ref[...]) * jnp.float32(scale)

    out = pl.pallas_call(
        body,
        out_shape=jax.ShapeDtypeStruct((1, 1), jnp.float32),
        in_specs=[pl.BlockSpec(memory_space=pltpu.VMEM)],
        out_specs=pl.BlockSpec(memory_space=pltpu.SMEM),
    )(x2d)
    return out[0, 0]


def kernel(predictions, targets):
    n_rows, n_cols = predictions.shape
    var = float(n_cols) * (float(n_cols) ** 2 - 1.0) / 12.0
    scale = -1.0 / ((var + 1e-8) * n_rows)
    sc = _make_sc_numerators(n_cols, n_rows, 32)
    nums = sc(predictions, targets)
    return _tc_reduce(nums, scale)


# speedup vs baseline: 3.3267x; 1.0908x over previous
import functools

import jax
import jax.numpy as jnp
from jax import lax
from jax.experimental import pallas as pl
from jax.experimental.pallas import tpu as pltpu
from jax.experimental.pallas import tpu_sc as plsc

_LANES = 16
_MIN_I32 = -2147483648
_RIC = 4


def _make_sc_numerators(n_cols, n_rows, num_workers, interpret=False):
    seglen = n_cols // _LANES
    rows_per = n_rows // num_workers
    tmask = seglen - 1
    tshift = seglen.bit_length() - 1
    radix = 256
    mu = (n_cols - 1) / 2.0

    mesh = plsc.VectorSubcoreMesh(core_axis_name="c", subcore_axis_name="s",
                                  num_cores=2, num_subcores=16)

    row_scratch = [
        pltpu.VMEM((n_cols,), jnp.float32),
        pltpu.VMEM((n_cols,), jnp.float32),
        pltpu.VMEM((n_cols,), jnp.int32),
        pltpu.VMEM((n_cols,), jnp.int32),
        pltpu.VMEM((n_cols,), jnp.int32),
        pltpu.VMEM((n_cols,), jnp.int32),
        pltpu.VMEM((radix * _LANES,), jnp.int32),
    ]

    @functools.partial(
        pl.kernel,
        out_type=jax.ShapeDtypeStruct((n_rows * _LANES,), jnp.float32),
        mesh=mesh,
        scratch_types=row_scratch * _RIC + [
            pltpu.VMEM((rows_per * _LANES,), jnp.float32),
            pltpu.SemaphoreType.DMA,
        ],
        compiler_params=pltpu.CompilerParams(needs_layout_passes=False),
        interpret=interpret,
    )
    def sc_kernel(pred_hbm, targ_hbm, out_hbm, *scr):
        bufps = [scr[j * 7 + 0] for j in range(_RIC)]
        bufts = [scr[j * 7 + 1] for j in range(_RIC)]
        kas = [scr[j * 7 + 2] for j in range(_RIC)]
        kbs = [scr[j * 7 + 3] for j in range(_RIC)]
        pas = [scr[j * 7 + 4] for j in range(_RIC)]
        pbs = [scr[j * 7 + 5] for j in range(_RIC)]
        hists = [scr[j * 7 + 6] for j in range(_RIC)]
        nums = scr[_RIC * 7]
        dma_sem = scr[_RIC * 7 + 1]

        wid = lax.axis_index("s") * 2 + lax.axis_index("c")
        lane = lax.iota(jnp.int32, _LANES)
        zeros16 = jnp.zeros((_LANES,), jnp.int32)
        ones16 = jnp.ones((_LANES,), jnp.int32)

        def skew(i):
            t = i & tmask
            s = jnp.right_shift(i, tshift)
            return (t << 4) + ((s + t) & 15)

        def key_transpose(srcs, dsts):
            def body(g, _):
                i = g * _LANES + lane
                dest = skew(i)
                xs = [srcs[j][pl.ds(g * _LANES, _LANES)] for j in range(_RIC)]
                keys = []
                for x in xs:
                    b = plsc.bitcast(x, jnp.int32)
                    b = jnp.where(x == 0.0, 0, b)
                    m = jnp.right_shift(b, 31)
                    keys.append((b ^ (m & 0x7FFFFFFF)) ^ _MIN_I32)
                for j in range(_RIC):
                    plsc.store_scatter(dsts[j], [dest], keys[j])
                return 0
            lax.fori_loop(0, seglen, body, 0, unroll=2)

        def radix_pass(shift, ksrcs, psrcs, kdsts, pdsts,
                       gen_payload=False, invert_to=None, dot=False):
            def clr(d, _):
                for j in range(_RIC):
                    hists[j][pl.ds(d * _LANES, _LANES)] = zeros16
                return 0
            lax.fori_loop(0, radix, clr, 0, unroll=8)

            def histo(t, _):
                s = (lane - t) & 15
                ks = [ksrcs[j][pl.ds(t * _LANES, _LANES)] for j in range(_RIC)]
                hidxs = [((jnp.right_shift(k, shift) & 255) << 4) + s for k in ks]
                for j in range(_RIC):
                    plsc.addupdate_scatter(hists[j], [hidxs[j]], ones16)
                return 0
            lax.fori_loop(0, seglen, histo, 0, unroll=4)

            def scan(d, tots):
                hs = [hists[j][pl.ds(d * _LANES, _LANES)] for j in range(_RIC)]
                incls = [plsc.cumsum(h) for h in hs]
                for j in range(_RIC):
                    hists[j][pl.ds(d * _LANES, _LANES)] = incls[j] - hs[j] + tots[j]
                return tuple(tots[j] + incls[j][_LANES - 1] for j in range(_RIC))
            lax.fori_loop(0, radix, scan, (jnp.int32(0),) * _RIC, unroll=4)

            def fetch(t):
                s = (lane - t) & 15
                ks = tuple(ksrcs[j][pl.ds(t * _LANES, _LANES)]
                           for j in range(_RIC))
                if gen_payload:
                    pays = (((s << tshift) + t),) * _RIC
                else:
                    pays = tuple(psrcs[j][pl.ds(t * _LANES, _LANES)]
                                 for j in range(_RIC))
                hidxs = tuple(((jnp.right_shift(k, shift) & 255) << 4) + s
                              for k in ks)
                return ks, pays, hidxs

            def rmw(hidxs):
                poss = [plsc.load_gather(hists[j], [hidxs[j]])
                        for j in range(_RIC)]
                for j in range(_RIC):
                    plsc.store_scatter(hists[j], [hidxs[j]], poss[j] + 1)
                return poss

            def emit(ks, pays, poss):
                if invert_to is not None:
                    dests = [skew(pays[j]) for j in range(_RIC)]
                    for j in range(_RIC):
                        plsc.store_scatter(invert_to[j], [dests[j]], poss[j])
                else:
                    dests = [skew(poss[j]) for j in range(_RIC)]
                    for j in range(_RIC):
                        plsc.store_scatter(kdsts[j], [dests[j]], ks[j])
                    for j in range(_RIC):
                        plsc.store_scatter(pdsts[j], [dests[j]], pays[j])

            if dot:
                def permute_dot(t, carry):
                    ks, pays, hidxs, accs = carry
                    poss = rmw(hidxs)
                    nxt = fetch(t + 1)
                    accs = tuple(
                        accs[j]
                        + (poss[j].astype(jnp.float32) - mu)
                        * (pays[j].astype(jnp.float32) - mu)
                        for j in range(_RIC))
                    return (*nxt, accs)
                carry0 = (*fetch(0),
                          (jnp.zeros((_LANES,), jnp.float32),) * _RIC)
                ks, pays, hidxs, accs = lax.fori_loop(
                    0, seglen - 1, permute_dot, carry0, unroll=2)
                poss = rmw(hidxs)
                return tuple(
                    accs[j]
                    + (poss[j].astype(jnp.float32) - mu)
                    * (pays[j].astype(jnp.float32) - mu)
                    for j in range(_RIC))

            def permute(t, carry):
                ks, pays, hidxs = carry
                poss = rmw(hidxs)
                nxt = fetch(t + 1)
                emit(ks, pays, poss)
                return nxt
            ks, pays, hidxs = lax.fori_loop(
                0, seglen - 1, permute, fetch(0), unroll=2)
            emit(ks, pays, rmw(hidxs))
            return None

        def do_pair(r, _):
            copies = []
            for j in range(_RIC):
                row = wid * rows_per + r * _RIC + j
                copies.append(pltpu.async_copy(pred_hbm.at[row], bufps[j], dma_sem))
                copies.append(pltpu.async_copy(targ_hbm.at[row], bufts[j], dma_sem))
            for c in copies:
                c.wait()
            key_transpose(bufps, kas)
            radix_pass(0, kas, pas, kbs, pbs, gen_payload=True)
            radix_pass(8, kbs, pbs, kas, pas)
            radix_pass(16, kas, pas, kbs, pbs)
            radix_pass(24, kbs, pbs, None, None, invert_to=pas)
            key_transpose(bufts, kas)
            radix_pass(0, kas, pas, kbs, pbs)
            radix_pass(8, kbs, pbs, kas, pas)
            radix_pass(16, kas, pas, kbs, pbs)
            accs = radix_pass(24, kbs, pbs, None, None, dot=True)
            for j in range(_RIC):
                nums[pl.ds((r * _RIC + j) * _LANES, _LANES)] = accs[j]
            return 0

        lax.fori_loop(0, rows_per // _RIC, do_pair, 0)
        pltpu.sync_copy(
            nums, out_hbm.at[pl.ds(wid * rows_per * _LANES, rows_per * _LANES)])

    return sc_kernel


def _tc_reduce(nums, scale):
    n = nums.shape[0]
    x2d = nums.reshape(n // 128, 128)

    def body(x_ref, o_ref):
        o_ref[0, 0] = jnp.sum(x_ref[...]) * jnp.float32(scale)

    out = pl.pallas_call(
        body,
        out_shape=jax.ShapeDtypeStruct((1, 1), jnp.float32),
        in_specs=[pl.BlockSpec(memory_space=pltpu.VMEM)],
        out_specs=pl.BlockSpec(memory_space=pltpu.SMEM),
    )(x2d)
    return out[0, 0]


def kernel(predictions, targets):
    n_rows, n_cols = predictions.shape
    var = float(n_cols) * (float(n_cols) ** 2 - 1.0) / 12.0
    scale = -1.0 / ((var + 1e-8) * n_rows)
    sc = _make_sc_numerators(n_cols, n_rows, 32)
    nums = sc(predictions, targets)
    return _tc_reduce(nums, scale)
